# baseline (speedup 1.0000x reference)
import functools

import jax
import jax.numpy as jnp
from jax import lax
from jax.experimental import pallas as pl
from jax.experimental.pallas import tpu as pltpu
from jax.experimental.pallas import tpu_sc as plsc

VOCAB = 1000000
SEQ_LEN = 200
EMBED_DIM = 64
BATCH = 4096

N_ROWS = BATCH * SEQ_LEN
NUM_WORKERS = 32
ROWS_PER_WORKER = N_ROWS // NUM_WORKERS
CHUNK = 800
CHUNKS_PER_WORKER = ROWS_PER_WORKER // CHUNK
GATHER_BATCH = 128
FULL_GATHERS = CHUNK // GATHER_BATCH
TAIL = CHUNK - FULL_GATHERS * GATHER_BATCH


def _body(inputs_hbm, token_hbm, pos_hbm, out_hbm, pos_sh, idx_v, rows_v, sem):
    sid = lax.axis_index("s")
    wid = sid * 2 + lax.axis_index("c")
    base = wid * ROWS_PER_WORKER

    @pl.when(sid == 0)
    def _():
        pltpu.sync_copy(pos_hbm, rows_v.at[pl.ds(0, SEQ_LEN)])
        pltpu.sync_copy(rows_v.at[pl.ds(0, SEQ_LEN)], pos_sh)

    plsc.subcore_barrier()

    def chunk_body(c, carry):
        start = base + c * CHUNK
        pltpu.sync_copy(inputs_hbm.at[pl.ds(start, CHUNK)], idx_v)
        for q in range(CHUNK // SEQ_LEN):
            pltpu.sync_copy(pos_sh, rows_v.at[pl.ds(q * SEQ_LEN, SEQ_LEN)])
        descs = []
        for g in range(FULL_GATHERS):
            descs.append(
                pltpu.async_copy(
                    token_hbm.at[idx_v.at[pl.ds(g * GATHER_BATCH, GATHER_BATCH)]],
                    rows_v.at[pl.ds(g * GATHER_BATCH, GATHER_BATCH)],
                    sem,
                    add=True,
                )
            )
        if TAIL:
            descs.append(
                pltpu.async_copy(
                    token_hbm.at[idx_v.at[pl.ds(FULL_GATHERS * GATHER_BATCH, TAIL)]],
                    rows_v.at[pl.ds(FULL_GATHERS * GATHER_BATCH, TAIL)],
                    sem,
                    add=True,
                )
            )
        for d in descs:
            d.wait()
        pltpu.sync_copy(rows_v, out_hbm.at[pl.ds(start, CHUNK)])
        return carry

    lax.fori_loop(0, CHUNKS_PER_WORKER, chunk_body, 0)


@jax.jit
def _run(inputs_flat, token_table, position_table):
    mesh = plsc.VectorSubcoreMesh(core_axis_name="c", subcore_axis_name="s")
    kern = pl.kernel(
        _body,
        out_type=jax.ShapeDtypeStruct((N_ROWS, EMBED_DIM), jnp.float32),
        mesh=mesh,
        scratch_types=[
            pltpu.VMEM_SHARED((SEQ_LEN, EMBED_DIM), jnp.float32),
            pltpu.VMEM((CHUNK,), jnp.int32),
            pltpu.VMEM((CHUNK, EMBED_DIM), jnp.float32),
            pltpu.SemaphoreType.DMA,
        ],
        compiler_params=pltpu.CompilerParams(use_tc_tiling_on_sc=False),
    )
    return kern(inputs_flat, token_table, position_table)


def kernel(inputs, token_table, position_table):
    inputs_flat = inputs.reshape(-1).astype(jnp.int32)
    out = _run(inputs_flat, token_table, position_table)
    return out.reshape(BATCH, SEQ_LEN, EMBED_DIM)


# speedup vs baseline: 1.0502x; 1.0502x over previous
import jax
import jax.numpy as jnp
from jax import lax
from jax.experimental import pallas as pl
from jax.experimental.pallas import tpu as pltpu
from jax.experimental.pallas import tpu_sc as plsc

VOCAB = 1000000
SEQ_LEN = 200
EMBED_DIM = 64
BATCH = 4096

N_ROWS = BATCH * SEQ_LEN
NUM_WORKERS = 32
ROWS_PER_WORKER = N_ROWS // NUM_WORKERS
CHUNK = 800
NUM_CHUNKS = ROWS_PER_WORKER // CHUNK
GATHER_BATCH = 128
FULL_GATHERS = CHUNK // GATHER_BATCH
TAIL = CHUNK - FULL_GATHERS * GATHER_BATCH


def _issue_gathers(token_hbm, idx_ref, rows_ref, sem):
    sizes = [GATHER_BATCH] * FULL_GATHERS + ([TAIL] if TAIL else [])
    off = 0
    for n in sizes:
        pltpu.async_copy(
            token_hbm.at[idx_ref.at[pl.ds(off, n)]],
            rows_ref.at[pl.ds(off, n)],
            sem,
            add=True,
        )
        off += n


def _body(inputs_hbm, token_hbm, pos_hbm, out_hbm,
          pos_sh, idx_v, rows_v, sem_g0, sem_g1, sem_o0, sem_o1):
    sid = lax.axis_index("s")
    wid = sid * 2 + lax.axis_index("c")
    base = wid * ROWS_PER_WORKER
    sems_g = (sem_g0, sem_g1)
    sems_o = (sem_o0, sem_o1)

    @pl.when(sid == 0)
    def _():
        pltpu.sync_copy(pos_hbm, rows_v.at[0, pl.ds(0, SEQ_LEN)])
        pltpu.sync_copy(rows_v.at[0, pl.ds(0, SEQ_LEN)], pos_sh)

    plsc.subcore_barrier()

    def prep_and_fire(c, b):
        start = base + c * CHUNK
        for q in range(CHUNK // SEQ_LEN):
            pltpu.sync_copy(pos_sh, rows_v.at[b, pl.ds(q * SEQ_LEN, SEQ_LEN)])
        pltpu.sync_copy(inputs_hbm.at[pl.ds(start, CHUNK)], idx_v.at[b])
        _issue_gathers(token_hbm, idx_v.at[b], rows_v.at[b], sems_g[b])

    def wait_gathers(b):
        pltpu.make_async_copy(
            out_hbm.at[pl.ds(0, CHUNK)], rows_v.at[b], sems_g[b]).wait()

    def fire_writeback(c, b):
        start = base + c * CHUNK
        pltpu.async_copy(rows_v.at[b], out_hbm.at[pl.ds(start, CHUNK)],
                         sems_o[b])

    def wait_writeback(c, b):
        start = base + c * CHUNK
        pltpu.make_async_copy(
            rows_v.at[b], out_hbm.at[pl.ds(start, CHUNK)], sems_o[b]).wait()

    prep_and_fire(0, 0)

    @pl.loop(1, NUM_CHUNKS)
    def _chunk(c):
        b = lax.rem(c, 2)

        @pl.when(b == 0)
        def _():
            @pl.when(c >= 2)
            def _():
                wait_writeback(c - 2, 0)
            prep_and_fire(c, 0)
            wait_gathers(1)
            fire_writeback(c - 1, 1)

        @pl.when(b == 1)
        def _():
            @pl.when(c >= 2)
            def _():
                wait_writeback(c - 2, 1)
            prep_and_fire(c, 1)
            wait_gathers(0)
            fire_writeback(c - 1, 0)

    last = NUM_CHUNKS - 1
    bl = last % 2
    wait_gathers(bl)
    fire_writeback(last, bl)
    wait_writeback(last - 1, 1 - bl)
    wait_writeback(last, bl)


@jax.jit
def _run(inputs_flat, token_table, position_table):
    mesh = plsc.VectorSubcoreMesh(core_axis_name="c", subcore_axis_name="s")
    kern = pl.kernel(
        _body,
        out_type=jax.ShapeDtypeStruct((N_ROWS, EMBED_DIM), jnp.float32),
        mesh=mesh,
        scratch_types=[
            pltpu.VMEM_SHARED((SEQ_LEN, EMBED_DIM), jnp.float32),
            pltpu.VMEM((2, CHUNK), jnp.int32),
            pltpu.VMEM((2, CHUNK, EMBED_DIM), jnp.float32),
            pltpu.SemaphoreType.DMA,
            pltpu.SemaphoreType.DMA,
            pltpu.SemaphoreType.DMA,
            pltpu.SemaphoreType.DMA,
        ],
        compiler_params=pltpu.CompilerParams(use_tc_tiling_on_sc=False),
    )
    return kern(inputs_flat, token_table, position_table)


def kernel(inputs, token_table, position_table):
    inputs_flat = inputs.reshape(-1).astype(jnp.int32)
    out = _run(inputs_flat, token_table, position_table)
    return out.reshape(BATCH, SEQ_LEN, EMBED_DIM)


# speedup vs baseline: 1.2555x; 1.1955x over previous
import jax
import jax.numpy as jnp
from jax import lax
from jax.experimental import pallas as pl
from jax.experimental.pallas import tpu as pltpu
from jax.experimental.pallas import tpu_sc as plsc

VOCAB = 1000000
SEQ_LEN = 200
EMBED_DIM = 64
BATCH = 4096
PAD_DIM = 128

N_ROWS = BATCH * SEQ_LEN
NUM_WORKERS = 32
ROWS_PER_WORKER = N_ROWS // NUM_WORKERS
BATCH_PER_CHUNK = 2
CHUNK = BATCH_PER_CHUNK * SEQ_LEN
NUM_CHUNKS = ROWS_PER_WORKER // CHUNK
IDX_STRIDE = 512
GATHER_BATCH = 128
FULL_GATHERS = CHUNK // GATHER_BATCH
TAIL = CHUNK - FULL_GATHERS * GATHER_BATCH


def _issue_gathers(token_hbm, idx_v, ibase, rows_ref, sem):
    sizes = [GATHER_BATCH] * FULL_GATHERS + ([TAIL] if TAIL else [])
    off = 0
    for n in sizes:
        pltpu.async_copy(
            token_hbm.at[idx_v.at[pl.ds(ibase + off, n)]],
            rows_ref.at[pl.ds(off, n)],
            sem,
            add=True,
        )
        off += n


def _body(inputs_hbm, token_hbm, pos_hbm, out_hbm,
          pos_sh, idx_v, rows_v, sem_g0, sem_g1, sem_o0, sem_o1):
    sid = lax.axis_index("s")
    wid = sid * 2 + lax.axis_index("c")
    base = wid * ROWS_PER_WORKER
    batch_base = wid * (ROWS_PER_WORKER // SEQ_LEN)
    sems_g = (sem_g0, sem_g1)
    sems_o = (sem_o0, sem_o1)

    @pl.when(sid == 0)
    def _():
        pltpu.sync_copy(pos_hbm, rows_v.at[0, pl.ds(0, SEQ_LEN)])
        pltpu.sync_copy(rows_v.at[0, pl.ds(0, SEQ_LEN)], pos_sh)

    plsc.subcore_barrier()

    def prep_and_fire(c, b):
        start = base + c * CHUNK
        for q in range(BATCH_PER_CHUNK):
            pltpu.sync_copy(pos_sh, rows_v.at[b, pl.ds(q * SEQ_LEN, SEQ_LEN)])
        pltpu.sync_copy(inputs_hbm.at[pl.ds(start, CHUNK)],
                        idx_v.at[pl.ds(b * IDX_STRIDE, CHUNK)])
        _issue_gathers(token_hbm, idx_v, b * IDX_STRIDE, rows_v.at[b],
                       sems_g[b])

    def wait_gathers(b):
        pltpu.make_async_copy(
            token_hbm.at[pl.ds(0, CHUNK)], rows_v.at[b], sems_g[b]).wait()

    def fire_writeback(c, b):
        brow = batch_base + c * BATCH_PER_CHUNK
        for r in range(BATCH_PER_CHUNK):
            pltpu.async_copy(
                rows_v.at[b, pl.ds(r * SEQ_LEN, SEQ_LEN)],
                out_hbm.at[brow + r],
                sems_o[b],
            )

    def wait_writeback(c, b):
        brow = batch_base + c * BATCH_PER_CHUNK
        for r in range(BATCH_PER_CHUNK):
            pltpu.make_async_copy(
                rows_v.at[b, pl.ds(r * SEQ_LEN, SEQ_LEN)],
                out_hbm.at[brow + r],
                sems_o[b],
            ).wait()

    prep_and_fire(0, 0)

    @pl.loop(1, NUM_CHUNKS)
    def _chunk(c):
        b = lax.rem(c, 2)

        @pl.when(b == 0)
        def _():
            @pl.when(c >= 2)
            def _():
                wait_writeback(c - 2, 0)
            prep_and_fire(c, 0)
            wait_gathers(1)
            fire_writeback(c - 1, 1)

        @pl.when(b == 1)
        def _():
            @pl.when(c >= 2)
            def _():
                wait_writeback(c - 2, 1)
            prep_and_fire(c, 1)
            wait_gathers(0)
            fire_writeback(c - 1, 0)

    last = NUM_CHUNKS - 1
    bl = last % 2
    wait_gathers(bl)
    fire_writeback(last, bl)
    wait_writeback(last - 1, 1 - bl)
    wait_writeback(last, bl)


@jax.jit
def _run(inputs_flat, token_padded, pos_padded):
    mesh = plsc.VectorSubcoreMesh(core_axis_name="c", subcore_axis_name="s")
    kern = pl.kernel(
        _body,
        out_type=jax.ShapeDtypeStruct((BATCH, SEQ_LEN, PAD_DIM), jnp.float32),
        mesh=mesh,
        scratch_types=[
            pltpu.VMEM_SHARED((SEQ_LEN, PAD_DIM), jnp.float32),
            pltpu.VMEM((2 * IDX_STRIDE,), jnp.int32),
            pltpu.VMEM((2, CHUNK, PAD_DIM), jnp.float32),
            pltpu.SemaphoreType.DMA,
            pltpu.SemaphoreType.DMA,
            pltpu.SemaphoreType.DMA,
            pltpu.SemaphoreType.DMA,
        ],
        compiler_params=pltpu.CompilerParams(use_tc_tiling_on_sc=True),
    )
    return kern(inputs_flat, token_padded, pos_padded)[:, :, :EMBED_DIM]


def kernel(inputs, token_table, position_table):
    inputs_flat = inputs.reshape(-1).astype(jnp.int32)
    token_padded = jnp.pad(token_table, ((0, 0), (0, PAD_DIM - EMBED_DIM)))
    pos_padded = jnp.pad(position_table, ((0, 0), (0, PAD_DIM - EMBED_DIM)))
    return _run(inputs_flat, token_padded, pos_padded)
